# baseline (speedup 1.0000x reference)
import functools

import jax
import jax.numpy as jnp
from jax import lax
from jax.experimental import pallas as pl
from jax.experimental.pallas import tpu as pltpu
from jax.experimental.pallas import tpu_sc as plsc

N = 10000
E = 160000
NP = 10240
CH = 128
NCH = 40
NW = 32
EP = NW * NCH * CH
RPT = NP // 16

@functools.lru_cache(maxsize=None)
def _mesh():
    return plsc.VectorSubcoreMesh(
        core_axis_name="c", subcore_axis_name="s",
        num_cores=2, num_subcores=16)


@functools.lru_cache(maxsize=None)
def _make_deg_kernel():
    @functools.partial(
        pl.kernel,
        out_type=jax.ShapeDtypeStruct((2, NP, 16), jnp.float32),
        mesh=_mesh(),
        compiler_params=pltpu.CompilerParams(use_tc_tiling_on_sc=False),
        scratch_types=[
            pltpu.VMEM((NCH, CH), jnp.int32),
            pltpu.VMEM((CH, 16), jnp.float32),
            pltpu.VMEM_SHARED((NP, 16), jnp.float32),
        ],
    )
    def deg_kernel(dst_hbm, ones_hbm, zblk_hbm, out_hbm, didx, ones_v, acc):
        c = lax.axis_index("c")
        s = lax.axis_index("s")
        w = c * 16 + s
        pltpu.sync_copy(zblk_hbm, acc.at[pl.ds(s * RPT, RPT)])
        pltpu.sync_copy(ones_hbm, ones_v)
        pltpu.sync_copy(dst_hbm.at[pl.ds(w * NCH, NCH)], didx)
        plsc.subcore_barrier()

        def body(j, carry):
            pltpu.sync_copy(ones_v, acc.at[didx.at[j]], add=True)
            return carry

        lax.fori_loop(0, NCH, body, 0)
        plsc.subcore_barrier()
        pltpu.sync_copy(acc.at[pl.ds(s * RPT, RPT)],
                        out_hbm.at[c, pl.ds(s * RPT, RPT)])

    return deg_kernel


NCH2 = 2 * NCH
K_SLOW = 40
K_FAST = NCH2 - K_SLOW
SLOW_CORE = 1
_OWN0 = K_SLOW if SLOW_CORE == 0 else K_FAST


def _ring(h_hbm, acc, sidx, didx, rows, gsems, ssems, nch):
    for b in range(3):
        pltpu.async_copy(h_hbm.at[sidx.at[b]], rows[b], gsems[b])

    def body(i, carry):
        for b in range(4):
            j = 4 * i + b
            bn = (b + 3) % 4
            pltpu.make_async_copy(
                h_hbm.at[sidx.at[j]], rows[b], gsems[b]).wait()
            pltpu.async_copy(rows[b], acc.at[didx.at[j]], ssems[b],
                             add=True)

            @pl.when(j + 3 < nch)
            def _():
                @pl.when(j >= 1)
                def _():
                    pltpu.make_async_copy(
                        rows[bn], acc.at[didx.at[j - 1]],
                        ssems[bn]).wait()
                pltpu.async_copy(
                    h_hbm.at[sidx.at[j + 3]], rows[bn], gsems[bn])
        return carry

    lax.fori_loop(0, nch // 4, body, 0)
    for b in range(4):
        pltpu.make_async_copy(
            rows[b], acc.at[didx.at[nch - 4 + b]], ssems[b]).wait()


@functools.lru_cache(maxsize=None)
def _make_agg_kernel(H, dtype=jnp.bfloat16):
    @functools.partial(
        pl.kernel,
        out_type=jax.ShapeDtypeStruct((2, NP, H), dtype),
        mesh=_mesh(),
        compiler_params=pltpu.CompilerParams(use_tc_tiling_on_sc=False),
        scratch_types=[
            pltpu.VMEM((K_FAST, CH), jnp.int32),
            pltpu.VMEM((K_FAST, CH), jnp.int32),
            [pltpu.VMEM((CH, H), dtype) for _ in range(4)],
            pltpu.VMEM_SHARED((NP, H), dtype),
            pltpu.VMEM_SHARED((N, H), dtype),
            [pltpu.SemaphoreType.DMA for _ in range(4)],
            [pltpu.SemaphoreType.DMA for _ in range(4)],
        ],
    )
    def agg_kernel(h_hbm, src_hbm, dst_hbm, zblk_hbm, out_hbm,
                   sidx, didx, rows, acc, tbl, gsems, ssems):
        c = lax.axis_index("c")
        s = lax.axis_index("s")
        base = s * NCH2 + c * _OWN0
        pltpu.sync_copy(zblk_hbm, acc.at[pl.ds(s * RPT, RPT)])
        pltpu.sync_copy(h_hbm.at[pl.ds(s * 625, 625)],
                        tbl.at[pl.ds(s * 625, 625)])

        @pl.when(c == SLOW_CORE)
        def _():
            pltpu.sync_copy(src_hbm.at[pl.ds(base, K_SLOW)],
                            sidx.at[pl.ds(0, K_SLOW)])
            pltpu.sync_copy(dst_hbm.at[pl.ds(base, K_SLOW)],
                            didx.at[pl.ds(0, K_SLOW)])

        @pl.when(c != SLOW_CORE)
        def _():
            pltpu.sync_copy(src_hbm.at[pl.ds(base, K_FAST)], sidx)
            pltpu.sync_copy(dst_hbm.at[pl.ds(base, K_FAST)], didx)

        plsc.subcore_barrier()

        @pl.when(c == SLOW_CORE)
        def _():
            _ring(tbl, acc, sidx, didx, rows, gsems, ssems, K_SLOW)

        @pl.when(c != SLOW_CORE)
        def _():
            _ring(tbl, acc, sidx, didx, rows, gsems, ssems, K_FAST)

        plsc.subcore_barrier()
        pltpu.sync_copy(acc.at[pl.ds(s * RPT, RPT)],
                        out_hbm.at[c, pl.ds(s * RPT, RPT)])

    return agg_kernel


def _deg_sc(dstp, ones16, zblk):
    return _make_deg_kernel()(dstp, ones16, zblk)


def _agg_sc(h, srcp, dstp, zblk):
    return _make_agg_kernel(h.shape[1])(h, srcp, dstp, zblk)


_RB = 10000


def _mm_body(x_ref, w_ref, o_ref):
    o_ref[...] = jnp.dot(x_ref[...], w_ref[...],
                         preferred_element_type=jnp.float32)


def _mm(x, w):
    n, d = x.shape
    h = w.shape[1]
    return pl.pallas_call(
        _mm_body,
        grid=(n // _RB,),
        in_specs=[pl.BlockSpec((_RB, d), lambda i: (i, 0)),
                  pl.BlockSpec((d, h), lambda i: (0, 0))],
        out_specs=pl.BlockSpec((_RB, h), lambda i: (i, 0)),
        out_shape=jax.ShapeDtypeStruct((n, h), jnp.float32),
    )(x, w)


def _dinv_body(s_ref, p_ref, dinv_ref, hs_ref):
    cnt = s_ref[0, :, 0:1] + s_ref[1, :, 0:1]
    dinv = lax.rsqrt(cnt + 1.0)
    dinv_ref[...] = dinv
    hs_ref[...] = (p_ref[...] * dinv).astype(jnp.bfloat16)


def _dinv_stage(cnt_part, p1):
    h = p1.shape[1]
    return pl.pallas_call(
        _dinv_body,
        grid=(N // _RB,),
        in_specs=[pl.BlockSpec((2, _RB, 16), lambda i: (0, i, 0)),
                  pl.BlockSpec((_RB, h), lambda i: (i, 0))],
        out_specs=[pl.BlockSpec((_RB, 1), lambda i: (i, 0)),
                   pl.BlockSpec((_RB, h), lambda i: (i, 0))],
        out_shape=[jax.ShapeDtypeStruct((N, 1), jnp.float32),
                   jax.ShapeDtypeStruct((N, h), jnp.bfloat16)],
    )(cnt_part, p1)


def _junction_body(s_ref, p_ref, dinv_ref, b_ref, w_ref, pn_ref, hs_ref):
    dinv = dinv_ref[...]
    agg = s_ref[0].astype(jnp.float32) + s_ref[1].astype(jnp.float32)
    g = agg * dinv + p_ref[...] * (dinv * dinv) + b_ref[...]
    g = jnp.maximum(g, 0.0)
    pn = jnp.dot(g, w_ref[...], preferred_element_type=jnp.float32)
    pn_ref[...] = pn
    hs_ref[...] = (pn * dinv).astype(jnp.bfloat16)


def _junction(s_part, p, dinv, b, w):
    hi = p.shape[1]
    ho = w.shape[1]
    return pl.pallas_call(
        _junction_body,
        grid=(N // _RB,),
        in_specs=[pl.BlockSpec((2, _RB, hi), lambda i: (0, i, 0)),
                  pl.BlockSpec((_RB, hi), lambda i: (i, 0)),
                  pl.BlockSpec((_RB, 1), lambda i: (i, 0)),
                  pl.BlockSpec((1, hi), lambda i: (0, 0)),
                  pl.BlockSpec((hi, ho), lambda i: (0, 0))],
        out_specs=[pl.BlockSpec((_RB, ho), lambda i: (i, 0)),
                   pl.BlockSpec((_RB, ho), lambda i: (i, 0))],
        out_shape=[jax.ShapeDtypeStruct((N, ho), jnp.float32),
                   jax.ShapeDtypeStruct((N, ho), jnp.bfloat16)],
    )(s_part, p, dinv, b, w)


def _final_body(s_ref, p_ref, dinv_ref, b_ref, w_ref, bfc_ref, o_ref):
    dinv = dinv_ref[...]
    agg = s_ref[0].astype(jnp.float32) + s_ref[1].astype(jnp.float32)
    g = agg * dinv + p_ref[...] * (dinv * dinv) + b_ref[...]
    g = jnp.maximum(g, 0.0)
    logits = jnp.dot(g, w_ref[...],
                     preferred_element_type=jnp.float32) + bfc_ref[...]
    m = jnp.max(logits, axis=1, keepdims=True)
    lse = jnp.log(jnp.sum(jnp.exp(logits - m), axis=1, keepdims=True)) + m
    o_ref[...] = logits - lse


def _final(s_part, p, dinv, b, wfc, bfc):
    hi = p.shape[1]
    cc = wfc.shape[1]
    return pl.pallas_call(
        _final_body,
        grid=(N // _RB,),
        in_specs=[pl.BlockSpec((2, _RB, hi), lambda i: (0, i, 0)),
                  pl.BlockSpec((_RB, hi), lambda i: (i, 0)),
                  pl.BlockSpec((_RB, 1), lambda i: (i, 0)),
                  pl.BlockSpec((1, hi), lambda i: (0, 0)),
                  pl.BlockSpec((hi, cc), lambda i: (0, 0)),
                  pl.BlockSpec((1, cc), lambda i: (0, 0))],
        out_specs=pl.BlockSpec((_RB, cc), lambda i: (i, 0)),
        out_shape=jax.ShapeDtypeStruct((N, cc), jnp.float32),
    )(s_part, p, dinv, b, wfc, bfc)


def _aggregate(h, srcp, dstp, zblk):
    return _agg_sc(h, srcp, dstp, zblk)


def kernel(x, edge_index, W1, b1, W2, b2, W3, b3, Wfc, bfc):
    src = edge_index[0]
    dst = edge_index[1]
    pad = EP - E
    srcp = jnp.concatenate(
        [src, jnp.zeros((pad,), jnp.int32)]).reshape(NW * NCH, CH)
    dstp = jnp.concatenate(
        [dst, jnp.full((pad,), N, jnp.int32)]).reshape(NW * NCH, CH)

    ones16 = jnp.ones((CH, 16), jnp.float32)
    z16 = jnp.zeros((RPT, 16), jnp.float32)
    z64 = jnp.zeros((RPT, 64), jnp.bfloat16)
    z128 = jnp.zeros((RPT, 128), jnp.bfloat16)

    cnt_part = _deg_sc(dstp, ones16, z16)
    p1 = _mm(x, W1)
    dinv, hs1 = _dinv_stage(cnt_part, p1)

    s1 = _aggregate(hs1, srcp, dstp, z64)
    p2, hs2 = _junction(s1, p1, dinv, b1.reshape(1, -1), W2)

    s2 = _aggregate(hs2, srcp, dstp, z128)
    p3, hs3 = _junction(s2, p2, dinv, b2.reshape(1, -1), W3)

    s3 = _aggregate(hs3, srcp, dstp, z64)
    out = _final(s3, p3, dinv, b3.reshape(1, -1), Wfc, bfc.reshape(1, -1))
    return out


# speedup vs baseline: 1.0215x; 1.0215x over previous
import functools

import jax
import jax.numpy as jnp
from jax import lax
from jax.experimental import pallas as pl
from jax.experimental.pallas import tpu as pltpu
from jax.experimental.pallas import tpu_sc as plsc

N = 10000
E = 160000
NP = 10240
CH = 128
NCH = 40
NW = 32
EP = NW * NCH * CH
RPT = NP // 16

@functools.lru_cache(maxsize=None)
def _mesh():
    return plsc.VectorSubcoreMesh(
        core_axis_name="c", subcore_axis_name="s",
        num_cores=2, num_subcores=16)


@functools.lru_cache(maxsize=None)
def _make_deg_kernel():
    @functools.partial(
        pl.kernel,
        out_type=jax.ShapeDtypeStruct((2, NP, 16), jnp.float32),
        mesh=_mesh(),
        compiler_params=pltpu.CompilerParams(use_tc_tiling_on_sc=False),
        scratch_types=[
            pltpu.VMEM((NCH, CH), jnp.int32),
            pltpu.VMEM((CH, 16), jnp.float32),
            pltpu.VMEM_SHARED((NP, 16), jnp.float32),
        ],
    )
    def deg_kernel(dst_hbm, ones_hbm, zblk_hbm, out_hbm, didx, ones_v, acc):
        c = lax.axis_index("c")
        s = lax.axis_index("s")
        w = c * 16 + s
        pltpu.sync_copy(zblk_hbm, acc.at[pl.ds(s * RPT, RPT)])
        pltpu.sync_copy(ones_hbm, ones_v)
        pltpu.sync_copy(dst_hbm.at[pl.ds(w * NCH, NCH)], didx)
        plsc.subcore_barrier()

        def body(j, carry):
            pltpu.sync_copy(ones_v, acc.at[didx.at[j]], add=True)
            return carry

        lax.fori_loop(0, NCH, body, 0)
        plsc.subcore_barrier()
        pltpu.sync_copy(acc.at[pl.ds(s * RPT, RPT)],
                        out_hbm.at[c, pl.ds(s * RPT, RPT)])

    return deg_kernel


NCH2 = 2 * NCH
K_SLOW = 40
K_FAST = NCH2 - K_SLOW
SLOW_CORE = 1
_OWN0 = K_SLOW if SLOW_CORE == 0 else K_FAST


_DEPTH = 4


def _ring(h_hbm, acc, sidx, didx, rows, gsems, ssems, nch):
    dp = _DEPTH
    for b in range(dp - 1):
        pltpu.async_copy(h_hbm.at[sidx.at[b]], rows[b], gsems[b])

    def body(i, carry):
        for b in range(dp):
            j = dp * i + b
            bn = (b + dp - 1) % dp
            pltpu.make_async_copy(
                h_hbm.at[sidx.at[j]], rows[b], gsems[b]).wait()
            pltpu.async_copy(rows[b], acc.at[didx.at[j]], ssems[b],
                             add=True)

            @pl.when(j + dp - 1 < nch)
            def _():
                @pl.when(j >= 1)
                def _():
                    pltpu.make_async_copy(
                        rows[bn], acc.at[didx.at[j - 1]],
                        ssems[bn]).wait()
                pltpu.async_copy(
                    h_hbm.at[sidx.at[j + dp - 1]], rows[bn], gsems[bn])
        return carry

    lax.fori_loop(0, nch // dp, body, 0)
    for b in range(dp):
        pltpu.make_async_copy(
            rows[b], acc.at[didx.at[nch - dp + b]], ssems[b]).wait()


@functools.lru_cache(maxsize=None)
def _make_agg_kernel(H, dtype=jnp.bfloat16):
    @functools.partial(
        pl.kernel,
        out_type=jax.ShapeDtypeStruct((2, NP, H), dtype),
        mesh=_mesh(),
        compiler_params=pltpu.CompilerParams(use_tc_tiling_on_sc=False),
        scratch_types=[
            pltpu.VMEM((K_FAST, CH), jnp.int32),
            pltpu.VMEM((K_FAST, CH), jnp.int32),
            [pltpu.VMEM((CH, H), dtype) for _ in range(_DEPTH)],
            pltpu.VMEM_SHARED((NP, H), dtype),
            pltpu.VMEM_SHARED((N, H), dtype),
            [pltpu.SemaphoreType.DMA for _ in range(_DEPTH)],
            [pltpu.SemaphoreType.DMA for _ in range(_DEPTH)],
        ],
    )
    def agg_kernel(h_hbm, src_hbm, dst_hbm, zblk_hbm, out_hbm,
                   sidx, didx, rows, acc, tbl, gsems, ssems):
        c = lax.axis_index("c")
        s = lax.axis_index("s")
        base = s * NCH2 + c * _OWN0
        pltpu.sync_copy(zblk_hbm, acc.at[pl.ds(s * RPT, RPT)])
        pltpu.sync_copy(h_hbm.at[pl.ds(s * 625, 625)],
                        tbl.at[pl.ds(s * 625, 625)])

        @pl.when(c == SLOW_CORE)
        def _():
            pltpu.sync_copy(src_hbm.at[pl.ds(base, K_SLOW)],
                            sidx.at[pl.ds(0, K_SLOW)])
            pltpu.sync_copy(dst_hbm.at[pl.ds(base, K_SLOW)],
                            didx.at[pl.ds(0, K_SLOW)])

        @pl.when(c != SLOW_CORE)
        def _():
            pltpu.sync_copy(src_hbm.at[pl.ds(base, K_FAST)], sidx)
            pltpu.sync_copy(dst_hbm.at[pl.ds(base, K_FAST)], didx)

        plsc.subcore_barrier()

        @pl.when(c == SLOW_CORE)
        def _():
            _ring(tbl, acc, sidx, didx, rows, gsems, ssems, K_SLOW)

        @pl.when(c != SLOW_CORE)
        def _():
            _ring(tbl, acc, sidx, didx, rows, gsems, ssems, K_FAST)

        plsc.subcore_barrier()
        pltpu.sync_copy(acc.at[pl.ds(s * RPT, RPT)],
                        out_hbm.at[c, pl.ds(s * RPT, RPT)])

    return agg_kernel


def _deg_sc(dstp, ones16, zblk):
    return _make_deg_kernel()(dstp, ones16, zblk)


def _agg_sc(h, srcp, dstp, zblk):
    return _make_agg_kernel(h.shape[1])(h, srcp, dstp, zblk)


_RB = 2000


def _mm_body(x_ref, w_ref, o_ref):
    o_ref[...] = jnp.dot(x_ref[...], w_ref[...],
                         preferred_element_type=jnp.float32)


def _mm(x, w):
    n, d = x.shape
    h = w.shape[1]
    return pl.pallas_call(
        _mm_body,
        grid=(n // _RB,),
        in_specs=[pl.BlockSpec((_RB, d), lambda i: (i, 0)),
                  pl.BlockSpec((d, h), lambda i: (0, 0))],
        out_specs=pl.BlockSpec((_RB, h), lambda i: (i, 0)),
        out_shape=jax.ShapeDtypeStruct((n, h), jnp.float32),
    )(x, w)


def _dinv_body(s_ref, p_ref, dinv_ref, hs_ref):
    cnt = s_ref[0, :, 0:1] + s_ref[1, :, 0:1]
    dinv = lax.rsqrt(cnt + 1.0)
    dinv_ref[...] = dinv
    hs_ref[...] = (p_ref[...] * dinv).astype(jnp.bfloat16)


def _dinv_stage(cnt_part, p1):
    h = p1.shape[1]
    return pl.pallas_call(
        _dinv_body,
        grid=(N // _RB,),
        in_specs=[pl.BlockSpec((2, _RB, 16), lambda i: (0, i, 0)),
                  pl.BlockSpec((_RB, h), lambda i: (i, 0))],
        out_specs=[pl.BlockSpec((_RB, 1), lambda i: (i, 0)),
                   pl.BlockSpec((_RB, h), lambda i: (i, 0))],
        out_shape=[jax.ShapeDtypeStruct((N, 1), jnp.float32),
                   jax.ShapeDtypeStruct((N, h), jnp.bfloat16)],
    )(cnt_part, p1)


def _junction_body(s_ref, p_ref, dinv_ref, b_ref, w_ref, pn_ref, hs_ref):
    dinv = dinv_ref[...]
    agg = s_ref[0].astype(jnp.float32) + s_ref[1].astype(jnp.float32)
    g = agg * dinv + p_ref[...] * (dinv * dinv) + b_ref[...]
    g = jnp.maximum(g, 0.0)
    pn = jnp.dot(g, w_ref[...], preferred_element_type=jnp.float32)
    pn_ref[...] = pn
    hs_ref[...] = (pn * dinv).astype(jnp.bfloat16)


def _junction(s_part, p, dinv, b, w):
    hi = p.shape[1]
    ho = w.shape[1]
    return pl.pallas_call(
        _junction_body,
        grid=(N // _RB,),
        in_specs=[pl.BlockSpec((2, _RB, hi), lambda i: (0, i, 0)),
                  pl.BlockSpec((_RB, hi), lambda i: (i, 0)),
                  pl.BlockSpec((_RB, 1), lambda i: (i, 0)),
                  pl.BlockSpec((1, hi), lambda i: (0, 0)),
                  pl.BlockSpec((hi, ho), lambda i: (0, 0))],
        out_specs=[pl.BlockSpec((_RB, ho), lambda i: (i, 0)),
                   pl.BlockSpec((_RB, ho), lambda i: (i, 0))],
        out_shape=[jax.ShapeDtypeStruct((N, ho), jnp.float32),
                   jax.ShapeDtypeStruct((N, ho), jnp.bfloat16)],
    )(s_part, p, dinv, b, w)


def _final_body(s_ref, p_ref, dinv_ref, b_ref, w_ref, bfc_ref, o_ref):
    dinv = dinv_ref[...]
    agg = s_ref[0].astype(jnp.float32) + s_ref[1].astype(jnp.float32)
    g = agg * dinv + p_ref[...] * (dinv * dinv) + b_ref[...]
    g = jnp.maximum(g, 0.0)
    logits = jnp.dot(g, w_ref[...],
                     preferred_element_type=jnp.float32) + bfc_ref[...]
    m = jnp.max(logits, axis=1, keepdims=True)
    lse = jnp.log(jnp.sum(jnp.exp(logits - m), axis=1, keepdims=True)) + m
    o_ref[...] = logits - lse


def _final(s_part, p, dinv, b, wfc, bfc):
    hi = p.shape[1]
    cc = wfc.shape[1]
    return pl.pallas_call(
        _final_body,
        grid=(N // _RB,),
        in_specs=[pl.BlockSpec((2, _RB, hi), lambda i: (0, i, 0)),
                  pl.BlockSpec((_RB, hi), lambda i: (i, 0)),
                  pl.BlockSpec((_RB, 1), lambda i: (i, 0)),
                  pl.BlockSpec((1, hi), lambda i: (0, 0)),
                  pl.BlockSpec((hi, cc), lambda i: (0, 0)),
                  pl.BlockSpec((1, cc), lambda i: (0, 0))],
        out_specs=pl.BlockSpec((_RB, cc), lambda i: (i, 0)),
        out_shape=jax.ShapeDtypeStruct((N, cc), jnp.float32),
    )(s_part, p, dinv, b, wfc, bfc)


def _aggregate(h, srcp, dstp, zblk):
    return _agg_sc(h, srcp, dstp, zblk)


def kernel(x, edge_index, W1, b1, W2, b2, W3, b3, Wfc, bfc):
    src = edge_index[0]
    dst = edge_index[1]
    pad = EP - E
    srcp = jnp.concatenate(
        [src, jnp.zeros((pad,), jnp.int32)]).reshape(NW * NCH, CH)
    dstp = jnp.concatenate(
        [dst, jnp.full((pad,), N, jnp.int32)]).reshape(NW * NCH, CH)

    ones16 = jnp.ones((CH, 16), jnp.float32)
    z16 = jnp.zeros((RPT, 16), jnp.float32)
    z64 = jnp.zeros((RPT, 64), jnp.bfloat16)
    z128 = jnp.zeros((RPT, 128), jnp.bfloat16)

    cnt_part = _deg_sc(dstp, ones16, z16)
    p1 = _mm(x, W1)
    dinv, hs1 = _dinv_stage(cnt_part, p1)

    s1 = _aggregate(hs1, srcp, dstp, z64)
    p2, hs2 = _junction(s1, p1, dinv, b1.reshape(1, -1), W2)

    s2 = _aggregate(hs2, srcp, dstp, z128)
    p3, hs3 = _junction(s2, p2, dinv, b2.reshape(1, -1), W3)

    s3 = _aggregate(hs3, srcp, dstp, z64)
    out = _final(s3, p3, dinv, b3.reshape(1, -1), Wfc, bfc.reshape(1, -1))
    return out
